# baseline (speedup 1.0000x reference)
import functools

import jax
import jax.numpy as jnp
import numpy as np
from jax import lax
from jax.experimental import pallas as pl
from jax.experimental.pallas import tpu as pltpu
from jax.experimental.pallas import tpu_sc as plsc

NUM_EMB = 1000000
D = 64
DP = 128
P_DROP = 0.1
NT = 16384
NH = 50

NC = 2
NS = 16
L = 16
NW = NC * NS
I_PER_W = NT // NW
CI = 256
CIS = CI + 1
NCH_I = I_PER_W // CI
N_CHUNKS = NH * NCH_I

BITS_WORDS = 32768
INV_KEEP = float(np.float32(1.0) / np.float32(1.0 - P_DROP))


@functools.partial(
    pl.kernel,
    mesh=plsc.VectorSubcoreMesh(core_axis_name="c", subcore_axis_name="s"),
    out_type=jax.ShapeDtypeStruct((NH, D, NT), jnp.float32),
    compiler_params=pltpu.CompilerParams(needs_layout_passes=False),
    scratch_types=[
        pltpu.VMEM((BITS_WORDS,), jnp.int32),
        pltpu.VMEM((CI,), jnp.int32),
        pltpu.VMEM((CI,), jnp.int32),
        pltpu.VMEM((CI,), jnp.int32),
        pltpu.VMEM((CI,), jnp.int32),
        pltpu.VMEM((CI,), jnp.float32),
        pltpu.VMEM((CI,), jnp.float32),
        pltpu.VMEM((CI, DP), jnp.float32),
        pltpu.VMEM((CI, DP), jnp.float32),
        pltpu.VMEM((D, CIS), jnp.float32),
        pltpu.VMEM((CI,), jnp.int32),
        pltpu.VMEM((CI,), jnp.int32),
        pltpu.SemaphoreType.DMA,
        pltpu.SemaphoreType.DMA,
    ],
)
def _emb_dropout_gather(tview_hbm, idx_hbm, bits_hbm, out_hbm,
                        bits_v, idx_a, idx_b, idx2_a, idx2_b,
                        scale_a, scale_b, rows_a, rows_b, outb_v,
                        cb_a, cb_b, sem0, sem1):
    wid = lax.axis_index("s") * NC + lax.axis_index("c")
    i_base = wid * I_PER_W
    pltpu.sync_copy(bits_hbm, bits_v)

    bufs = ((idx_a, idx2_a, scale_a, rows_a, cb_a, sem0),
            (idx_b, idx2_b, scale_b, rows_b, cb_b, sem1))
    iota = lax.iota(jnp.int32, L)

    def chunk_coords(t):
        j = t // NCH_I
        i0 = i_base + (t % NCH_I) * CI
        return j, i0

    def issue(t, buf):
        idx_v, idx2_v, scale_v, rows_v, cb_v, sem = bufs[buf]
        j, i0 = chunk_coords(t)
        pltpu.sync_copy(idx_hbm.at[j, pl.ds(i0, CI)], idx_v)

        @plsc.parallel_loop(0, CI // L, unroll=4)
        def halve_body(g):
            idx2_v[pl.ds(g * L, L)] = lax.shift_right_logical(
                idx_v[pl.ds(g * L, L)], 1)

        pltpu.async_copy(tview_hbm.at[idx2_v], rows_v, sem)

        @plsc.parallel_loop(0, CI // L, unroll=4)
        def scale_body(g):
            idx16 = idx_v[pl.ds(g * L, L)]
            word = plsc.load_gather(bits_v, [lax.shift_right_logical(idx16, 5)])
            bit = lax.bitwise_and(
                lax.shift_right_logical(word, lax.bitwise_and(idx16, 31)), 1)
            cb_v[pl.ds(g * L, L)] = (
                lax.bitwise_and(idx16, 1) * D + bit * 256)

    def wait(buf):
        _, idx2_v, _, rows_v, _, sem = bufs[buf]
        pltpu.make_async_copy(tview_hbm.at[idx2_v], rows_v, sem).wait()

    def process(t, buf):
        _, _, scale_v, rows_v, cb_v, _ = bufs[buf]

        @plsc.parallel_loop(0, CI, unroll=16)
        def row_body(p):
            pvec = jnp.zeros((L,), jnp.int32) + p
            pw = plsc.load_gather(cb_v, [pvec])
            s16 = (lax.shift_right_logical(pw, 8).astype(jnp.float32)
                   * INV_KEEP)
            cbase16 = lax.bitwise_and(pw, 255)
            for cb in range(D // L):
                v = plsc.load_gather(rows_v, [pvec, cbase16 + (cb * L) + iota])
                plsc.store_scatter(outb_v, [cb * L + iota, pvec], v * s16)
        j, i0 = chunk_coords(t)
        pltpu.sync_copy(outb_v.at[:, pl.ds(0, CI)],
                        out_hbm.at[j, :, pl.ds(i0, CI)])

    issue(0, 0)

    def pair_body(p, carry):
        t0 = 2 * p
        issue(t0 + 1, 1)
        wait(0)
        process(t0, 0)

        @pl.when(p + 1 < N_CHUNKS // 2)
        def _():
            issue(t0 + 2, 0)

        wait(1)
        process(t0 + 1, 1)
        return carry

    lax.fori_loop(0, N_CHUNKS // 2, pair_body, 0)


def _pack_keep_bits():
    keep = jax.random.bernoulli(
        jax.random.key(42), 1.0 - P_DROP, (NUM_EMB, 1))
    kb = keep[:, 0]
    kb = jnp.pad(kb, (0, BITS_WORDS * 32 - NUM_EMB))
    kw = kb.reshape(BITS_WORDS, 32).astype(jnp.uint32)
    shifts = jnp.arange(32, dtype=jnp.uint32)[None, :]
    words_u = jnp.sum(kw << shifts, axis=1, dtype=jnp.uint32)
    return lax.bitcast_convert_type(words_u, jnp.int32)


def kernel(words, table):
    bits = _pack_keep_bits()
    tview = table.reshape(NUM_EMB // 2, DP)
    wt = words.T
    out_k = _emb_dropout_gather(tview, wt, bits)
    return out_k.transpose(2, 0, 1)


# speedup vs baseline: 1.0326x; 1.0326x over previous
import functools

import jax
import jax.numpy as jnp
import numpy as np
from jax import lax
from jax.experimental import pallas as pl
from jax.experimental.pallas import tpu as pltpu
from jax.experimental.pallas import tpu_sc as plsc

NUM_EMB = 1000000
D = 64
DP = 128
P_DROP = 0.1
NT = 16384
NH = 50

NC = 2
NS = 16
L = 16
NW = NC * NS
I_PER_W = NT // NW
CI = 128
CIS = CI + 1
NCH_I = I_PER_W // CI
N_CHUNKS = NH * NCH_I

BITS_WORDS = 31264
INV_KEEP = float(np.float32(1.0) / np.float32(1.0 - P_DROP))


@functools.partial(
    pl.kernel,
    mesh=plsc.VectorSubcoreMesh(core_axis_name="c", subcore_axis_name="s"),
    out_type=jax.ShapeDtypeStruct((NH, D, NT), jnp.float32),
    compiler_params=pltpu.CompilerParams(needs_layout_passes=False),
    scratch_types=[
        pltpu.VMEM((BITS_WORDS,), jnp.int32),
        pltpu.VMEM((CI,), jnp.int32),
        pltpu.VMEM((CI,), jnp.int32),
        pltpu.VMEM((CI, DP), jnp.float32),
        pltpu.VMEM((CI, DP), jnp.float32),
        pltpu.VMEM((D, CIS), jnp.float32),
        pltpu.VMEM((D, CIS), jnp.float32),
        pltpu.VMEM((CI,), jnp.int32),
        pltpu.VMEM((CI,), jnp.int32),
        pltpu.SemaphoreType.DMA,
        pltpu.SemaphoreType.DMA,
        pltpu.SemaphoreType.DMA,
        pltpu.SemaphoreType.DMA,
    ],
)
def _emb_dropout_gather(tview_hbm, idx_hbm, bits_hbm, out_hbm,
                        bits_v, idx_a, idx_b, rows_a, rows_b,
                        outb_a, outb_b, cb_a, cb_b,
                        sem0, sem1, osem0, osem1):
    wid = lax.axis_index("s") * NC + lax.axis_index("c")
    i_base = wid * I_PER_W
    pltpu.sync_copy(bits_hbm, bits_v)

    bufs = ((idx_a, rows_a, outb_a, cb_a, sem0, osem0),
            (idx_b, rows_b, outb_b, cb_b, sem1, osem1))
    iota = lax.iota(jnp.int32, L)

    def chunk_coords(t):
        j = t // NCH_I
        i0 = i_base + (t % NCH_I) * CI
        return j, i0

    def issue(t, buf):
        idx_v, rows_v, _, cb_v, sem, _ = bufs[buf]
        j, i0 = chunk_coords(t)
        pltpu.sync_copy(idx_hbm.at[j, pl.ds(i0, CI)], idx_v)

        @plsc.parallel_loop(0, CI // L, unroll=4)
        def scale_body(g):
            idx16 = idx_v[pl.ds(g * L, L)]
            word = plsc.load_gather(bits_v, [lax.shift_right_logical(idx16, 5)])
            bit = lax.bitwise_and(
                lax.shift_right_logical(word, lax.bitwise_and(idx16, 31)), 1)
            cb_v[pl.ds(g * L, L)] = (
                lax.bitwise_and(idx16, 1) * D + bit * 256)
            idx_v[pl.ds(g * L, L)] = lax.shift_right_logical(idx16, 1)

        pltpu.async_copy(tview_hbm.at[idx_v], rows_v, sem)

    def wait(buf):
        idx_v, rows_v, _, _, sem, _ = bufs[buf]
        pltpu.make_async_copy(tview_hbm.at[idx_v], rows_v, sem).wait()

    def out_copy(t, buf, start):
        _, _, outb_v, _, _, osem = bufs[buf]
        j, i0 = chunk_coords(t)
        cp = pltpu.make_async_copy(outb_v.at[:, pl.ds(0, CI)],
                                   out_hbm.at[j, :, pl.ds(i0, CI)], osem)
        cp.start() if start else cp.wait()

    def process(t, buf):
        _, rows_v, outb_v, cb_v, _, _ = bufs[buf]

        @pl.when(t >= 2)
        def _():
            out_copy(t - 2, buf, start=False)

        @plsc.parallel_loop(0, CI, unroll=16)
        def row_body(p):
            pvec = jnp.zeros((L,), jnp.int32) + p
            pw = plsc.load_gather(cb_v, [pvec])
            s16 = (lax.shift_right_logical(pw, 8).astype(jnp.float32)
                   * INV_KEEP)
            cbase16 = lax.bitwise_and(pw, 255)
            for cb in range(D // L):
                v = plsc.load_gather(rows_v, [pvec, cbase16 + (cb * L) + iota])
                plsc.store_scatter(outb_v, [cb * L + iota, pvec], v * s16)

        out_copy(t, buf, start=True)

    issue(0, 0)

    def pair_body(p, carry):
        t0 = 2 * p
        issue(t0 + 1, 1)
        wait(0)
        process(t0, 0)

        @pl.when(p + 1 < N_CHUNKS // 2)
        def _():
            issue(t0 + 2, 0)

        wait(1)
        process(t0 + 1, 1)
        return carry

    lax.fori_loop(0, N_CHUNKS // 2, pair_body, 0)
    out_copy(N_CHUNKS - 2, 0, start=False)
    out_copy(N_CHUNKS - 1, 1, start=False)


def _pack_keep_bits():
    keep = jax.random.bernoulli(
        jax.random.key(42), 1.0 - P_DROP, (NUM_EMB, 1))
    kb = keep[:, 0]
    kb = jnp.pad(kb, (0, BITS_WORDS * 32 - NUM_EMB))
    kw = kb.reshape(BITS_WORDS, 32).astype(jnp.uint32)
    shifts = jnp.arange(32, dtype=jnp.uint32)[None, :]
    words_u = jnp.sum(kw << shifts, axis=1, dtype=jnp.uint32)
    return lax.bitcast_convert_type(words_u, jnp.int32)


def kernel(words, table):
    bits = _pack_keep_bits()
    tview = table.reshape(NUM_EMB // 2, DP)
    wt = words.T
    out_k = _emb_dropout_gather(tview, wt, bits)
    return out_k.transpose(2, 0, 1)


# speedup vs baseline: 1.0353x; 1.0026x over previous
import functools

import jax
import jax.numpy as jnp
import numpy as np
from jax import lax
from jax.experimental import pallas as pl
from jax.experimental.pallas import tpu as pltpu
from jax.experimental.pallas import tpu_sc as plsc

NUM_EMB = 1000000
D = 64
DP = 128
P_DROP = 0.1
NT = 16384
NH = 50

NC = 2
NS = 16
L = 16
NW = NC * NS
I_PER_W = NT // NW
CI = 128
CIS = CI + 1
NCH_I = I_PER_W // CI
N_CHUNKS = NH * NCH_I

BITS_WORDS = 31264
INV_KEEP = float(np.float32(1.0) / np.float32(1.0 - P_DROP))


@functools.partial(
    pl.kernel,
    mesh=plsc.VectorSubcoreMesh(core_axis_name="c", subcore_axis_name="s"),
    out_type=jax.ShapeDtypeStruct((NH, D, NT), jnp.float32),
    compiler_params=pltpu.CompilerParams(needs_layout_passes=False),
    scratch_types=[
        pltpu.VMEM((BITS_WORDS,), jnp.int32),
        pltpu.VMEM((CI,), jnp.int32),
        pltpu.VMEM((CI,), jnp.int32),
        pltpu.VMEM((CI, DP), jnp.float32),
        pltpu.VMEM((CI, DP), jnp.float32),
        pltpu.VMEM((D, CIS), jnp.float32),
        pltpu.VMEM((D, CIS), jnp.float32),
        pltpu.VMEM((CI,), jnp.int32),
        pltpu.VMEM((CI,), jnp.int32),
        pltpu.SemaphoreType.DMA,
        pltpu.SemaphoreType.DMA,
        pltpu.SemaphoreType.DMA,
        pltpu.SemaphoreType.DMA,
    ],
)
def _emb_dropout_gather(tview_hbm, idx_hbm, bits_hbm, out_hbm,
                        bits_v, idx_a, idx_b, rows_a, rows_b,
                        outb_a, outb_b, cb_a, cb_b,
                        sem0, sem1, osem0, osem1):
    wid = lax.axis_index("s") * NC + lax.axis_index("c")
    i_base = wid * I_PER_W
    pltpu.sync_copy(bits_hbm, bits_v)

    bufs = ((idx_a, rows_a, outb_a, cb_a, sem0, osem0),
            (idx_b, rows_b, outb_b, cb_b, sem1, osem1))
    iota = lax.iota(jnp.int32, L)

    def chunk_coords(t):
        j = t // NCH_I
        i0 = i_base + (t % NCH_I) * CI
        return j, i0

    def issue(t, buf):
        idx_v, rows_v, _, cb_v, sem, _ = bufs[buf]
        j, i0 = chunk_coords(t)
        pltpu.sync_copy(idx_hbm.at[j, pl.ds(i0, CI)], idx_v)

        @plsc.parallel_loop(0, CI // L, unroll=4)
        def scale_body(g):
            idx16 = idx_v[pl.ds(g * L, L)]
            word = plsc.load_gather(bits_v, [lax.shift_right_logical(idx16, 5)])
            bit = lax.bitwise_and(
                lax.shift_right_logical(word, lax.bitwise_and(idx16, 31)), 1)
            cb_v[pl.ds(g * L, L)] = (
                lax.bitwise_and(idx16, 1) * D + bit * 256)
            idx_v[pl.ds(g * L, L)] = lax.shift_right_logical(idx16, 1)

        pltpu.async_copy(tview_hbm.at[idx_v], rows_v, sem)

    def wait(buf):
        idx_v, rows_v, _, _, sem, _ = bufs[buf]
        pltpu.make_async_copy(tview_hbm.at[idx_v], rows_v, sem).wait()

    def out_copy(t, buf, start):
        _, _, outb_v, _, _, osem = bufs[buf]
        j, i0 = chunk_coords(t)
        cp = pltpu.make_async_copy(outb_v.at[:, pl.ds(0, CI)],
                                   out_hbm.at[j, :, pl.ds(i0, CI)], osem)
        cp.start() if start else cp.wait()

    def process(t, buf):
        _, rows_v, outb_v, cb_v, _, _ = bufs[buf]

        @pl.when(t >= 2)
        def _():
            out_copy(t - 2, buf, start=False)

        @plsc.parallel_loop(0, CI, unroll=8)
        def row_body(p):
            pvec = jnp.zeros((L,), jnp.int32) + p
            pw = plsc.load_gather(cb_v, [pvec])
            s16 = (lax.shift_right_logical(pw, 8).astype(jnp.float32)
                   * INV_KEEP)
            cbase16 = lax.bitwise_and(pw, 255)
            for cb in range(D // L):
                v = plsc.load_gather(rows_v, [pvec, cbase16 + (cb * L) + iota])
                plsc.store_scatter(outb_v, [cb * L + iota, pvec], v * s16)

        out_copy(t, buf, start=True)

    issue(0, 0)

    def pair_body(p, carry):
        t0 = 2 * p
        issue(t0 + 1, 1)
        wait(0)
        process(t0, 0)

        @pl.when(p + 1 < N_CHUNKS // 2)
        def _():
            issue(t0 + 2, 0)

        wait(1)
        process(t0 + 1, 1)
        return carry

    lax.fori_loop(0, N_CHUNKS // 2, pair_body, 0)
    out_copy(N_CHUNKS - 2, 0, start=False)
    out_copy(N_CHUNKS - 1, 1, start=False)


def _pack_keep_bits():
    keep = jax.random.bernoulli(
        jax.random.key(42), 1.0 - P_DROP, (NUM_EMB, 1))
    kb = keep[:, 0]
    kb = jnp.pad(kb, (0, BITS_WORDS * 32 - NUM_EMB))
    kw = kb.reshape(BITS_WORDS, 32).astype(jnp.uint32)
    shifts = jnp.arange(32, dtype=jnp.uint32)[None, :]
    words_u = jnp.sum(kw << shifts, axis=1, dtype=jnp.uint32)
    return lax.bitcast_convert_type(words_u, jnp.int32)


def kernel(words, table):
    bits = _pack_keep_bits()
    tview = table.reshape(NUM_EMB // 2, DP)
    wt = words.T
    out_k = _emb_dropout_gather(tview, wt, bits)
    return out_k.transpose(2, 0, 1)
